# baseline (device time: 17721 ns/iter reference)
import jax
import jax.numpy as jnp
from jax import lax
from jax.experimental import pallas as pl
from jax.experimental.pallas import tpu as pltpu

N_DEV = 4


def kernel(x, router_W, route_idx, expert_W):
    del router_W
    n, d = x.shape
    e_per, _, h = expert_W.shape
    hc = h // N_DEV

    def body(x_ref, idx_ref, w_ref, out_ref, stage_ref, rs_comm_ref,
             ag_stage_ref, ag_comm_ref, rs_send_sems, rs_recv_sems,
             ag_send_sems, ag_recv_sems):
        my_pos = lax.axis_index("i")

        barrier_sem = pltpu.get_barrier_semaphore()
        for o in range(1, N_DEV):
            pl.semaphore_signal(
                barrier_sem, inc=1,
                device_id=((my_pos + o) % N_DEV,),
                device_id_type=pl.DeviceIdType.MESH,
            )

        route = idx_ref[:, :]
        xv = x_ref[:, :]
        xm = [
            jnp.where(route == my_pos * e_per + e, xv, 0.0).astype(jnp.bfloat16)
            for e in range(e_per)
        ]

        pl.semaphore_wait(barrier_sem, N_DEV - 1)

        for c in range(N_DEV):
            chunk = jnp.zeros((n, hc), dtype=jnp.float32)
            for e in range(e_per):
                chunk = chunk + jnp.dot(
                    xm[e], w_ref[e, :, c * hc:(c + 1) * hc].astype(jnp.bfloat16),
                    preferred_element_type=jnp.float32,
                )
            stage_ref[c, :, :] = chunk.astype(jnp.bfloat16)
            o = (c - my_pos) % N_DEV

            @pl.when(o != 0)
            def _(c=c, o=o):
                rdma = pltpu.make_async_remote_copy(
                    src_ref=stage_ref.at[c],
                    dst_ref=rs_comm_ref.at[o - 1],
                    send_sem=rs_send_sems.at[o - 1],
                    recv_sem=rs_recv_sems.at[o - 1],
                    device_id=(c,),
                    device_id_type=pl.DeviceIdType.MESH,
                )
                rdma.start()

        final = stage_ref[my_pos].astype(jnp.float32)
        for o in range(1, N_DEV):
            recv = pltpu.make_async_remote_copy(
                src_ref=stage_ref.at[0],
                dst_ref=rs_comm_ref.at[o - 1],
                send_sem=rs_send_sems.at[o - 1],
                recv_sem=rs_recv_sems.at[o - 1],
                device_id=((my_pos + o) % N_DEV,),
                device_id_type=pl.DeviceIdType.MESH,
            )
            recv.wait_recv()
            final = final + rs_comm_ref[o - 1, :, :].astype(jnp.float32)
        ag_stage_ref[:, :] = final.astype(jnp.bfloat16)

        for o in range(1, N_DEV):
            rdma = pltpu.make_async_remote_copy(
                src_ref=ag_stage_ref,
                dst_ref=ag_comm_ref.at[o - 1],
                send_sem=ag_send_sems.at[o - 1],
                recv_sem=ag_recv_sems.at[o - 1],
                device_id=((my_pos + o) % N_DEV,),
                device_id_type=pl.DeviceIdType.MESH,
            )
            rdma.start()

        out_ref[:, pl.ds(my_pos * hc, hc)] = final

        for o in range(1, N_DEV):
            recv = pltpu.make_async_remote_copy(
                src_ref=ag_stage_ref,
                dst_ref=ag_comm_ref.at[o - 1],
                send_sem=ag_send_sems.at[o - 1],
                recv_sem=ag_recv_sems.at[o - 1],
                device_id=((my_pos + o) % N_DEV,),
                device_id_type=pl.DeviceIdType.MESH,
            )
            recv.wait_recv()
            p = (my_pos - o) % N_DEV
            out_ref[:, pl.ds(p * hc, hc)] = (
                ag_comm_ref[o - 1, :, :].astype(jnp.float32)
            )

        for k in range(N_DEV - 1):
            for sems in (rs_send_sems, ag_send_sems):
                drain = pltpu.make_async_remote_copy(
                    src_ref=ag_stage_ref,
                    dst_ref=ag_comm_ref.at[k],
                    send_sem=sems.at[k],
                    recv_sem=ag_recv_sems.at[k],
                    device_id=(0,),
                    device_id_type=pl.DeviceIdType.MESH,
                )
                drain.wait_send()

    return pl.pallas_call(
        body,
        out_shape=jax.ShapeDtypeStruct((n, h), jnp.float32),
        in_specs=[
            pl.BlockSpec(memory_space=pltpu.VMEM),
            pl.BlockSpec(memory_space=pltpu.VMEM),
            pl.BlockSpec(memory_space=pltpu.VMEM),
        ],
        out_specs=pl.BlockSpec(memory_space=pltpu.VMEM),
        scratch_shapes=[
            pltpu.VMEM((N_DEV, n, hc), jnp.bfloat16),
            pltpu.VMEM((N_DEV - 1, n, hc), jnp.bfloat16),
            pltpu.VMEM((n, hc), jnp.bfloat16),
            pltpu.VMEM((N_DEV - 1, n, hc), jnp.bfloat16),
            pltpu.SemaphoreType.DMA((N_DEV - 1,)),
            pltpu.SemaphoreType.DMA((N_DEV - 1,)),
            pltpu.SemaphoreType.DMA((N_DEV - 1,)),
            pltpu.SemaphoreType.DMA((N_DEV - 1,)),
        ],
        compiler_params=pltpu.CompilerParams(collective_id=0),
    )(x, route_idx, expert_W)


# device time: 16963 ns/iter; 1.0447x vs baseline; 1.0447x over previous
import jax
import jax.numpy as jnp
from jax import lax
from jax.experimental import pallas as pl
from jax.experimental.pallas import tpu as pltpu

N_DEV = 4
S = 2


def kernel(x, router_W, route_idx, expert_W):
    del router_W
    n, d = x.shape
    e_per, _, h = expert_W.shape
    hc = h // N_DEV
    nr = n // S

    def body(x_ref, idx_ref, w_ref, out_ref, stage_ref, rs_comm_ref,
             ag_stage_ref, ag_comm_ref, rs_send_sems, rs_recv_sems,
             ag_send_sems, ag_recv_sems):
        my_pos = lax.axis_index("i")

        barrier_sem = pltpu.get_barrier_semaphore()
        for o in range(1, N_DEV):
            pl.semaphore_signal(
                barrier_sem, inc=1,
                device_id=((my_pos + o) % N_DEV,),
                device_id_type=pl.DeviceIdType.MESH,
            )

        route = idx_ref[:, :]
        xv = x_ref[:, :]
        xm = [
            jnp.where(route == my_pos * e_per + e, xv, 0.0).astype(jnp.bfloat16)
            for e in range(e_per)
        ]

        pl.semaphore_wait(barrier_sem, N_DEV - 1)

        for c in range(N_DEV):
            chunk = jnp.zeros((n, hc), dtype=jnp.float32)
            for e in range(e_per):
                chunk = chunk + jnp.dot(
                    xm[e], w_ref[e, :, c * hc:(c + 1) * hc].astype(jnp.bfloat16),
                    preferred_element_type=jnp.float32,
                )
            stage_ref[c, :, :] = chunk.astype(jnp.bfloat16)
            o = (c - my_pos) % N_DEV

            @pl.when(o != 0)
            def _(c=c, o=o):
                for s in range(S):
                    rdma = pltpu.make_async_remote_copy(
                        src_ref=stage_ref.at[c, pl.ds(s * nr, nr), :],
                        dst_ref=rs_comm_ref.at[o - 1, pl.ds(s * nr, nr), :],
                        send_sem=rs_send_sems.at[o - 1, s],
                        recv_sem=rs_recv_sems.at[o - 1, s],
                        device_id=(c,),
                        device_id_type=pl.DeviceIdType.MESH,
                    )
                    rdma.start()

        for s in range(S):
            rows = pl.ds(s * nr, nr)
            for o in range(1, N_DEV):
                recv = pltpu.make_async_remote_copy(
                    src_ref=stage_ref.at[0, rows, :],
                    dst_ref=rs_comm_ref.at[o - 1, rows, :],
                    send_sem=rs_send_sems.at[o - 1, s],
                    recv_sem=rs_recv_sems.at[o - 1, s],
                    device_id=((my_pos + o) % N_DEV,),
                    device_id_type=pl.DeviceIdType.MESH,
                )
                recv.wait_recv()
            final = stage_ref[my_pos, rows, :].astype(jnp.float32)
            for o in range(1, N_DEV):
                final = final + rs_comm_ref[o - 1, rows, :].astype(jnp.float32)
            ag_stage_ref[rows, :] = final.astype(jnp.bfloat16)
            for o in range(1, N_DEV):
                rdma = pltpu.make_async_remote_copy(
                    src_ref=ag_stage_ref.at[rows, :],
                    dst_ref=ag_comm_ref.at[o - 1, rows, :],
                    send_sem=ag_send_sems.at[o - 1, s],
                    recv_sem=ag_recv_sems.at[o - 1, s],
                    device_id=((my_pos + o) % N_DEV,),
                    device_id_type=pl.DeviceIdType.MESH,
                )
                rdma.start()
            out_ref[rows, pl.ds(my_pos * hc, hc)] = final

        for s in range(S):
            rows = pl.ds(s * nr, nr)
            for o in range(1, N_DEV):
                recv = pltpu.make_async_remote_copy(
                    src_ref=ag_stage_ref.at[rows, :],
                    dst_ref=ag_comm_ref.at[o - 1, rows, :],
                    send_sem=ag_send_sems.at[o - 1, s],
                    recv_sem=ag_recv_sems.at[o - 1, s],
                    device_id=((my_pos + o) % N_DEV,),
                    device_id_type=pl.DeviceIdType.MESH,
                )
                recv.wait_recv()
                p = (my_pos - o) % N_DEV
                out_ref[rows, pl.ds(p * hc, hc)] = (
                    ag_comm_ref[o - 1, rows, :].astype(jnp.float32)
                )

        for k in range(N_DEV - 1):
            for s in range(S):
                for sems in (rs_send_sems, ag_send_sems):
                    drain = pltpu.make_async_remote_copy(
                        src_ref=ag_stage_ref.at[pl.ds(s * nr, nr), :],
                        dst_ref=ag_comm_ref.at[k, pl.ds(s * nr, nr), :],
                        send_sem=sems.at[k, s],
                        recv_sem=ag_recv_sems.at[k, s],
                        device_id=(0,),
                        device_id_type=pl.DeviceIdType.MESH,
                    )
                    drain.wait_send()

    return pl.pallas_call(
        body,
        out_shape=jax.ShapeDtypeStruct((n, h), jnp.float32),
        in_specs=[
            pl.BlockSpec(memory_space=pltpu.VMEM),
            pl.BlockSpec(memory_space=pltpu.VMEM),
            pl.BlockSpec(memory_space=pltpu.VMEM),
        ],
        out_specs=pl.BlockSpec(memory_space=pltpu.VMEM),
        scratch_shapes=[
            pltpu.VMEM((N_DEV, n, hc), jnp.bfloat16),
            pltpu.VMEM((N_DEV - 1, n, hc), jnp.bfloat16),
            pltpu.VMEM((n, hc), jnp.bfloat16),
            pltpu.VMEM((N_DEV - 1, n, hc), jnp.bfloat16),
            pltpu.SemaphoreType.DMA((N_DEV - 1, S)),
            pltpu.SemaphoreType.DMA((N_DEV - 1, S)),
            pltpu.SemaphoreType.DMA((N_DEV - 1, S)),
            pltpu.SemaphoreType.DMA((N_DEV - 1, S)),
        ],
        compiler_params=pltpu.CompilerParams(collective_id=0),
    )(x, route_idx, expert_W)


# device time: 15853 ns/iter; 1.1178x vs baseline; 1.0700x over previous
import jax
import jax.numpy as jnp
from jax import lax
from jax.experimental import pallas as pl
from jax.experimental.pallas import tpu as pltpu

N_DEV = 4
S = 2


def kernel(x, router_W, route_idx, expert_W):
    del router_W
    n, d = x.shape
    e_per, _, h = expert_W.shape
    hc = h // N_DEV
    nr = n // S

    def body(x_ref, idx_ref, w_ref, out_ref, stage_ref, rs_comm_ref,
             ag_stage_ref, ag_comm_ref, rs_send_sems, rs_recv_sems,
             ag_send_sems, ag_recv_sems):
        my_pos = lax.axis_index("i")

        barrier_sem = pltpu.get_barrier_semaphore()
        for o in range(1, N_DEV):
            pl.semaphore_signal(
                barrier_sem, inc=1,
                device_id=((my_pos + o) % N_DEV,),
                device_id_type=pl.DeviceIdType.MESH,
            )

        route = idx_ref[:, :]
        xv = x_ref[:, :]
        xm = [
            jnp.where(route == my_pos * e_per + e, xv, 0.0).astype(jnp.bfloat16)
            for e in range(e_per)
        ]

        for c in range(N_DEV):
            chunk = jnp.zeros((n, hc), dtype=jnp.float32)
            for e in range(e_per):
                chunk = chunk + jnp.dot(
                    xm[e], w_ref[e, :, c * hc:(c + 1) * hc].astype(jnp.bfloat16),
                    preferred_element_type=jnp.float32,
                )
            stage_ref[c, :, :] = chunk.astype(jnp.bfloat16)

        pl.semaphore_wait(barrier_sem, N_DEV - 1)

        for c in range(N_DEV):
            o = (c - my_pos) % N_DEV

            @pl.when(o != 0)
            def _(c=c, o=o):
                for s in range(S):
                    rdma = pltpu.make_async_remote_copy(
                        src_ref=stage_ref.at[c, pl.ds(s * nr, nr), :],
                        dst_ref=rs_comm_ref.at[o - 1, pl.ds(s * nr, nr), :],
                        send_sem=rs_send_sems.at[o - 1, s],
                        recv_sem=rs_recv_sems.at[o - 1, s],
                        device_id=(c,),
                        device_id_type=pl.DeviceIdType.MESH,
                    )
                    rdma.start()

        for s in range(S):
            rows = pl.ds(s * nr, nr)
            for o in range(1, N_DEV):
                recv = pltpu.make_async_remote_copy(
                    src_ref=stage_ref.at[0, rows, :],
                    dst_ref=rs_comm_ref.at[o - 1, rows, :],
                    send_sem=rs_send_sems.at[o - 1, s],
                    recv_sem=rs_recv_sems.at[o - 1, s],
                    device_id=((my_pos + o) % N_DEV,),
                    device_id_type=pl.DeviceIdType.MESH,
                )
                recv.wait_recv()
            final = stage_ref[my_pos, rows, :].astype(jnp.float32)
            for o in range(1, N_DEV):
                final = final + rs_comm_ref[o - 1, rows, :].astype(jnp.float32)
            ag_stage_ref[rows, :] = final.astype(jnp.bfloat16)
            for o in range(1, N_DEV):
                rdma = pltpu.make_async_remote_copy(
                    src_ref=ag_stage_ref.at[rows, :],
                    dst_ref=ag_comm_ref.at[o - 1, rows, :],
                    send_sem=ag_send_sems.at[o - 1, s],
                    recv_sem=ag_recv_sems.at[o - 1, s],
                    device_id=((my_pos + o) % N_DEV,),
                    device_id_type=pl.DeviceIdType.MESH,
                )
                rdma.start()
            out_ref[rows, pl.ds(my_pos * hc, hc)] = final

        for s in range(S):
            rows = pl.ds(s * nr, nr)
            for o in range(1, N_DEV):
                recv = pltpu.make_async_remote_copy(
                    src_ref=ag_stage_ref.at[rows, :],
                    dst_ref=ag_comm_ref.at[o - 1, rows, :],
                    send_sem=ag_send_sems.at[o - 1, s],
                    recv_sem=ag_recv_sems.at[o - 1, s],
                    device_id=((my_pos + o) % N_DEV,),
                    device_id_type=pl.DeviceIdType.MESH,
                )
                recv.wait_recv()
                p = (my_pos - o) % N_DEV
                out_ref[rows, pl.ds(p * hc, hc)] = (
                    ag_comm_ref[o - 1, rows, :].astype(jnp.float32)
                )

        for k in range(N_DEV - 1):
            for s in range(S):
                for sems in (rs_send_sems, ag_send_sems):
                    drain = pltpu.make_async_remote_copy(
                        src_ref=ag_stage_ref.at[pl.ds(s * nr, nr), :],
                        dst_ref=ag_comm_ref.at[k, pl.ds(s * nr, nr), :],
                        send_sem=sems.at[k, s],
                        recv_sem=ag_recv_sems.at[k, s],
                        device_id=(0,),
                        device_id_type=pl.DeviceIdType.MESH,
                    )
                    drain.wait_send()

    return pl.pallas_call(
        body,
        out_shape=jax.ShapeDtypeStruct((n, h), jnp.float32),
        in_specs=[
            pl.BlockSpec(memory_space=pltpu.VMEM),
            pl.BlockSpec(memory_space=pltpu.VMEM),
            pl.BlockSpec(memory_space=pltpu.VMEM),
        ],
        out_specs=pl.BlockSpec(memory_space=pltpu.VMEM),
        scratch_shapes=[
            pltpu.VMEM((N_DEV, n, hc), jnp.bfloat16),
            pltpu.VMEM((N_DEV - 1, n, hc), jnp.bfloat16),
            pltpu.VMEM((n, hc), jnp.bfloat16),
            pltpu.VMEM((N_DEV - 1, n, hc), jnp.bfloat16),
            pltpu.SemaphoreType.DMA((N_DEV - 1, S)),
            pltpu.SemaphoreType.DMA((N_DEV - 1, S)),
            pltpu.SemaphoreType.DMA((N_DEV - 1, S)),
            pltpu.SemaphoreType.DMA((N_DEV - 1, S)),
        ],
        compiler_params=pltpu.CompilerParams(collective_id=0),
    )(x, route_idx, expert_W)


# device time: 15802 ns/iter; 1.1214x vs baseline; 1.0032x over previous
import jax
import jax.numpy as jnp
from jax import lax
from jax.experimental import pallas as pl
from jax.experimental.pallas import tpu as pltpu

N_DEV = 4
S = 2


def kernel(x, router_W, route_idx, expert_W):
    del router_W
    n, d = x.shape
    e_per, _, h = expert_W.shape
    hc = h // N_DEV
    nr = n // S

    def body(x_hbm_ref, idx_ref, w_hbm_ref, out_ref, x_ref, w_ref,
             stage_ref, rs_comm_ref, ag_stage_ref, ag_comm_ref,
             in_sems, out_sems, rs_send_sems, rs_recv_sems,
             ag_send_sems, ag_recv_sems):
        my_pos = lax.axis_index("i")

        barrier_sem = pltpu.get_barrier_semaphore()
        for o in range(1, N_DEV):
            pl.semaphore_signal(
                barrier_sem, inc=1,
                device_id=((my_pos + o) % N_DEV,),
                device_id_type=pl.DeviceIdType.MESH,
            )

        x_dma = pltpu.make_async_copy(x_hbm_ref, x_ref, in_sems.at[0])
        w_dma = pltpu.make_async_copy(w_hbm_ref, w_ref, in_sems.at[1])
        x_dma.start()
        w_dma.start()

        route = idx_ref[:, :]
        x_dma.wait()
        xv = x_ref[:, :]
        xm = [
            jnp.where(route == my_pos * e_per + e, xv, 0.0).astype(jnp.bfloat16)
            for e in range(e_per)
        ]

        w_dma.wait()
        for c in range(N_DEV):
            chunk = jnp.zeros((n, hc), dtype=jnp.float32)
            for e in range(e_per):
                chunk = chunk + jnp.dot(
                    xm[e], w_ref[e, :, c * hc:(c + 1) * hc].astype(jnp.bfloat16),
                    preferred_element_type=jnp.float32,
                )
            stage_ref[c, :, :] = chunk.astype(jnp.bfloat16)

        pl.semaphore_wait(barrier_sem, N_DEV - 1)

        for c in range(N_DEV):
            o = (c - my_pos) % N_DEV

            @pl.when(o != 0)
            def _(c=c, o=o):
                for s in range(S):
                    rdma = pltpu.make_async_remote_copy(
                        src_ref=stage_ref.at[c, pl.ds(s * nr, nr), :],
                        dst_ref=rs_comm_ref.at[o - 1, pl.ds(s * nr, nr), :],
                        send_sem=rs_send_sems.at[o - 1, s],
                        recv_sem=rs_recv_sems.at[o - 1, s],
                        device_id=(c,),
                        device_id_type=pl.DeviceIdType.MESH,
                    )
                    rdma.start()

        for s in range(S):
            rows = pl.ds(s * nr, nr)
            for o in range(1, N_DEV):
                recv = pltpu.make_async_remote_copy(
                    src_ref=stage_ref.at[0, rows, :],
                    dst_ref=rs_comm_ref.at[o - 1, rows, :],
                    send_sem=rs_send_sems.at[o - 1, s],
                    recv_sem=rs_recv_sems.at[o - 1, s],
                    device_id=((my_pos + o) % N_DEV,),
                    device_id_type=pl.DeviceIdType.MESH,
                )
                recv.wait_recv()
            final = stage_ref[my_pos, rows, :].astype(jnp.float32)
            for o in range(1, N_DEV):
                final = final + rs_comm_ref[o - 1, rows, :].astype(jnp.float32)
            ag_stage_ref[rows, :] = final.astype(jnp.bfloat16)
            for o in range(1, N_DEV):
                rdma = pltpu.make_async_remote_copy(
                    src_ref=ag_stage_ref.at[rows, :],
                    dst_ref=ag_comm_ref.at[o - 1, rows, :],
                    send_sem=ag_send_sems.at[o - 1, s],
                    recv_sem=ag_recv_sems.at[o - 1, s],
                    device_id=((my_pos + o) % N_DEV,),
                    device_id_type=pl.DeviceIdType.MESH,
                )
                rdma.start()
            pltpu.make_async_copy(
                ag_stage_ref.at[rows, :],
                out_ref.at[rows, pl.ds(my_pos * hc, hc)],
                out_sems.at[my_pos, s],
            ).start()

        for s in range(S):
            rows = pl.ds(s * nr, nr)
            for o in range(1, N_DEV):
                recv = pltpu.make_async_remote_copy(
                    src_ref=ag_stage_ref.at[rows, :],
                    dst_ref=ag_comm_ref.at[o - 1, rows, :],
                    send_sem=ag_send_sems.at[o - 1, s],
                    recv_sem=ag_recv_sems.at[o - 1, s],
                    device_id=((my_pos + o) % N_DEV,),
                    device_id_type=pl.DeviceIdType.MESH,
                )
                recv.wait_recv()
                p = (my_pos - o) % N_DEV
                pltpu.make_async_copy(
                    ag_comm_ref.at[o - 1, rows, :],
                    out_ref.at[rows, pl.ds(p * hc, hc)],
                    out_sems.at[p, s],
                ).start()

        for s in range(S):
            rows = pl.ds(s * nr, nr)
            for p in range(N_DEV):
                pltpu.make_async_copy(
                    ag_stage_ref.at[rows, :],
                    out_ref.at[rows, pl.ds(p * hc, hc)],
                    out_sems.at[p, s],
                ).wait()

        for k in range(N_DEV - 1):
            for s in range(S):
                for sems in (rs_send_sems, ag_send_sems):
                    drain = pltpu.make_async_remote_copy(
                        src_ref=ag_stage_ref.at[pl.ds(s * nr, nr), :],
                        dst_ref=ag_comm_ref.at[k, pl.ds(s * nr, nr), :],
                        send_sem=sems.at[k, s],
                        recv_sem=ag_recv_sems.at[k, s],
                        device_id=(0,),
                        device_id_type=pl.DeviceIdType.MESH,
                    )
                    drain.wait_send()

    return pl.pallas_call(
        body,
        out_shape=jax.ShapeDtypeStruct((n, h), jnp.bfloat16),
        in_specs=[
            pl.BlockSpec(memory_space=pl.ANY),
            pl.BlockSpec(memory_space=pltpu.VMEM),
            pl.BlockSpec(memory_space=pl.ANY),
        ],
        out_specs=pl.BlockSpec(memory_space=pl.ANY),
        scratch_shapes=[
            pltpu.VMEM((n, d), jnp.float32),
            pltpu.VMEM((e_per, d, h), jnp.float32),
            pltpu.VMEM((N_DEV, n, hc), jnp.bfloat16),
            pltpu.VMEM((N_DEV - 1, n, hc), jnp.bfloat16),
            pltpu.VMEM((n, hc), jnp.bfloat16),
            pltpu.VMEM((N_DEV - 1, n, hc), jnp.bfloat16),
            pltpu.SemaphoreType.DMA((2,)),
            pltpu.SemaphoreType.DMA((N_DEV, S)),
            pltpu.SemaphoreType.DMA((N_DEV - 1, S)),
            pltpu.SemaphoreType.DMA((N_DEV - 1, S)),
            pltpu.SemaphoreType.DMA((N_DEV - 1, S)),
            pltpu.SemaphoreType.DMA((N_DEV - 1, S)),
        ],
        compiler_params=pltpu.CompilerParams(collective_id=0),
    )(x, route_idx, expert_W)


# device time: 15750 ns/iter; 1.1251x vs baseline; 1.0033x over previous
import jax
import jax.numpy as jnp
from jax import lax
from jax.experimental import pallas as pl
from jax.experimental.pallas import tpu as pltpu

N_DEV = 4
S = 2


def kernel(x, router_W, route_idx, expert_W):
    del router_W
    n, d = x.shape
    e_per, _, h = expert_W.shape
    hc = h // N_DEV
    nr = n // S

    def body(x_hbm_ref, idx_ref, w_hbm_ref, out_ref, x_ref, w_ref,
             stage_ref, rs_comm_ref, ag_stage_ref, ag_comm_ref,
             in_sems, out_sems, rs_send_sems, rs_recv_sems,
             ag_send_sems, ag_recv_sems):
        my_pos = lax.axis_index("i")

        barrier_sem = pltpu.get_barrier_semaphore()
        for o in range(1, N_DEV):
            pl.semaphore_signal(
                barrier_sem, inc=1,
                device_id=((my_pos + o) % N_DEV,),
                device_id_type=pl.DeviceIdType.MESH,
            )

        x_dma = pltpu.make_async_copy(x_hbm_ref, x_ref, in_sems.at[0])
        w_dma = pltpu.make_async_copy(w_hbm_ref, w_ref, in_sems.at[1])
        x_dma.start()
        w_dma.start()

        route = idx_ref[:, :]
        x_dma.wait()
        xv = x_ref[:, :]
        xm = [
            jnp.where(route == my_pos * e_per + e, xv, 0.0).astype(jnp.bfloat16)
            for e in range(e_per)
        ]

        w_dma.wait()
        for c in range(N_DEV):
            chunk = jnp.zeros((n, hc), dtype=jnp.float32)
            for e in range(e_per):
                chunk = chunk + jnp.dot(
                    xm[e], w_ref[e, :, c * hc:(c + 1) * hc].astype(jnp.bfloat16),
                    preferred_element_type=jnp.float32,
                )
            stage_ref[c, :, :] = chunk.astype(jnp.bfloat16)

        pl.semaphore_wait(barrier_sem, N_DEV - 1)

        for c in range(N_DEV):
            o = (c - my_pos) % N_DEV

            @pl.when(o != 0)
            def _(c=c, o=o):
                for s in range(S):
                    rdma = pltpu.make_async_remote_copy(
                        src_ref=stage_ref.at[c, pl.ds(s * nr, nr), :],
                        dst_ref=rs_comm_ref.at[o - 1, pl.ds(s * nr, nr), :],
                        send_sem=rs_send_sems.at[o - 1, s],
                        recv_sem=rs_recv_sems.at[o - 1, s],
                        device_id=(c,),
                        device_id_type=pl.DeviceIdType.MESH,
                    )
                    rdma.start()

        for s in range(S):
            rows = pl.ds(s * nr, nr)
            for o in range(1, N_DEV):
                recv = pltpu.make_async_remote_copy(
                    src_ref=stage_ref.at[0, rows, :],
                    dst_ref=rs_comm_ref.at[o - 1, rows, :],
                    send_sem=rs_send_sems.at[o - 1, s],
                    recv_sem=rs_recv_sems.at[o - 1, s],
                    device_id=((my_pos + o) % N_DEV,),
                    device_id_type=pl.DeviceIdType.MESH,
                )
                recv.wait_recv()
            final = stage_ref[my_pos, rows, :].astype(jnp.float32)
            for o in range(1, N_DEV):
                final = final + rs_comm_ref[o - 1, rows, :].astype(jnp.float32)
            ag_stage_ref[rows, :] = final.astype(jnp.bfloat16)
            for o in range(1, N_DEV):
                rdma = pltpu.make_async_remote_copy(
                    src_ref=ag_stage_ref.at[rows, :],
                    dst_ref=ag_comm_ref.at[o - 1, rows, :],
                    send_sem=ag_send_sems.at[o - 1, s],
                    recv_sem=ag_recv_sems.at[o - 1, s],
                    device_id=((my_pos + o) % N_DEV,),
                    device_id_type=pl.DeviceIdType.MESH,
                )
                rdma.start()
            pltpu.make_async_copy(
                ag_stage_ref.at[rows, :],
                out_ref.at[rows, pl.ds(my_pos * hc, hc)],
                out_sems.at[my_pos, s],
            ).start()

        for s in range(S):
            rows = pl.ds(s * nr, nr)
            for o in range(1, N_DEV):
                recv = pltpu.make_async_remote_copy(
                    src_ref=ag_stage_ref.at[rows, :],
                    dst_ref=ag_comm_ref.at[o - 1, rows, :],
                    send_sem=ag_send_sems.at[o - 1, s],
                    recv_sem=ag_recv_sems.at[o - 1, s],
                    device_id=((my_pos + o) % N_DEV,),
                    device_id_type=pl.DeviceIdType.MESH,
                )
                recv.wait_recv()
                p = (my_pos - o) % N_DEV
                pltpu.make_async_copy(
                    ag_comm_ref.at[o - 1, rows, :],
                    out_ref.at[rows, pl.ds(p * hc, hc)],
                    out_sems.at[p, s],
                ).start()

        for s in range(S):
            rows = pl.ds(s * nr, nr)
            for p in range(N_DEV):
                pltpu.make_async_copy(
                    ag_stage_ref.at[rows, :],
                    out_ref.at[rows, pl.ds(p * hc, hc)],
                    out_sems.at[p, s],
                ).wait()

        for k in range(N_DEV - 1):
            for s in range(S):
                for sems in (rs_send_sems, ag_send_sems):
                    drain = pltpu.make_async_remote_copy(
                        src_ref=ag_stage_ref.at[pl.ds(s * nr, nr), :],
                        dst_ref=ag_comm_ref.at[k, pl.ds(s * nr, nr), :],
                        send_sem=sems.at[k, s],
                        recv_sem=ag_recv_sems.at[k, s],
                        device_id=(0,),
                        device_id_type=pl.DeviceIdType.MESH,
                    )
                    drain.wait_send()

    return pl.pallas_call(
        body,
        out_shape=jax.ShapeDtypeStruct((n, h), jnp.bfloat16),
        in_specs=[
            pl.BlockSpec(memory_space=pltpu.MemorySpace.HBM),
            pl.BlockSpec(memory_space=pltpu.VMEM),
            pl.BlockSpec(memory_space=pltpu.MemorySpace.HBM),
        ],
        out_specs=pl.BlockSpec(memory_space=pltpu.MemorySpace.HBM),
        scratch_shapes=[
            pltpu.VMEM((n, d), jnp.float32),
            pltpu.VMEM((e_per, d, h), jnp.float32),
            pltpu.VMEM((N_DEV, n, hc), jnp.bfloat16),
            pltpu.VMEM((N_DEV - 1, n, hc), jnp.bfloat16),
            pltpu.VMEM((n, hc), jnp.bfloat16),
            pltpu.VMEM((N_DEV - 1, n, hc), jnp.bfloat16),
            pltpu.SemaphoreType.DMA((2,)),
            pltpu.SemaphoreType.DMA((N_DEV, S)),
            pltpu.SemaphoreType.DMA((N_DEV - 1, S)),
            pltpu.SemaphoreType.DMA((N_DEV - 1, S)),
            pltpu.SemaphoreType.DMA((N_DEV - 1, S)),
            pltpu.SemaphoreType.DMA((N_DEV - 1, S)),
        ],
        compiler_params=pltpu.CompilerParams(collective_id=0),
    )(x, route_idx, expert_W)
